# Initial kernel scaffold; baseline (speedup 1.0000x reference)
import jax
import jax.numpy as jnp
from jax.experimental import pallas as pl


def kernel(x, params, edge_index):
    raise NotImplementedError("write your pallas kernel here")


# speedup vs baseline: 3.4460x; 3.4460x over previous
import functools

import jax
import jax.numpy as jnp
from jax import lax
from jax.experimental import pallas as pl
from jax.experimental.pallas import tpu as pltpu
from jax.experimental.pallas import tpu_sc as plsc

N_NODES = 10000
NP = 10240
NC, NS = 2, 16
NW = NC * NS
CH = 128
RPS = NP // NS

_f32 = jnp.float32


def _sds(shape):
    return jax.ShapeDtypeStruct(shape, _f32)


def _mesh():
    return plsc.VectorSubcoreMesh(core_axis_name="c", subcore_axis_name="s")


def _sc_gather2(E, D1, D2):
    per_w = E // NW
    steps = per_w // CH

    @functools.partial(
        pl.kernel,
        mesh=_mesh(),
        out_type=(_sds((E, D1)), _sds((E, D2))),
        scratch_types=[
            pltpu.VMEM((CH,), jnp.int32),
            pltpu.VMEM((CH, D1), _f32),
            pltpu.VMEM((CH,), jnp.int32),
            pltpu.VMEM((CH, D2), _f32),
            pltpu.SemaphoreType.DMA,
            pltpu.SemaphoreType.DMA,
        ],
    )
    def gk(t1, i1, t2, i2, o1, o2, iv1, rv1, iv2, rv2, sm1, sm2):
        wid = lax.axis_index("s") * NC + lax.axis_index("c")
        base = wid * per_w

        def body(j, carry):
            off = base + j * CH
            pltpu.sync_copy(i1.at[pl.ds(off, CH)], iv1)
            pltpu.sync_copy(i2.at[pl.ds(off, CH)], iv2)
            c1 = pltpu.async_copy(t1.at[iv1], rv1, sm1)
            c2 = pltpu.async_copy(t2.at[iv2], rv2, sm2)
            c1.wait()
            c2.wait()
            pltpu.sync_copy(rv1, o1.at[pl.ds(off, CH)])
            pltpu.sync_copy(rv2, o2.at[pl.ds(off, CH)])
            return carry

        lax.fori_loop(0, steps, body, 0)

    return gk


def _sc_scatter_gat(E):
    per_s = E // NS
    steps = per_s // CH

    @functools.partial(
        pl.kernel,
        mesh=_mesh(),
        out_type=_sds((NC, NP, 128)),
        scratch_types=[
            pltpu.VMEM((CH,), jnp.int32),
            pltpu.VMEM((CH, 128), _f32),
            pltpu.VMEM_SHARED((NP, 128), _f32),
        ],
    )
    def sk(P, si, z, out, iv, rv, acc):
        cid = lax.axis_index("c")
        sid = lax.axis_index("s")
        r0 = sid * RPS
        pltpu.sync_copy(z.at[pl.ds(r0, RPS)], acc.at[pl.ds(r0, RPS)])
        plsc.subcore_barrier()
        ibase = sid * per_s
        pbase = cid * E + ibase

        def body(j, carry):
            off = j * CH
            pltpu.sync_copy(si.at[pl.ds(ibase + off, CH)], iv)
            pltpu.sync_copy(P.at[pl.ds(pbase + off, CH)], rv)
            pltpu.sync_copy(rv, acc.at[iv], add=True)
            return carry

        lax.fori_loop(0, steps, body, 0)
        plsc.subcore_barrier()
        pltpu.sync_copy(acc.at[pl.ds(r0, RPS)], out.at[cid, pl.ds(r0, RPS)])

    return sk


def _sc_seg_msg(E):
    per_s = E // NS
    steps = per_s // CH

    @functools.partial(
        pl.kernel,
        mesh=_mesh(),
        out_type=_sds((E, 256)),
        scratch_types=[
            pltpu.VMEM((CH,), jnp.int32),
            pltpu.VMEM((CH, 128), _f32),
            pltpu.VMEM_SHARED((NP, 128), _f32),
            pltpu.SemaphoreType.DMA,
        ],
    )
    def mk(v, si, gi, z, msg, iv, rv, acc, sem):
        cid = lax.axis_index("c")
        sid = lax.axis_index("s")
        c0 = cid * 128
        r0 = sid * RPS
        pltpu.sync_copy(z.at[pl.ds(r0, RPS)], acc.at[pl.ds(r0, RPS)])
        plsc.subcore_barrier()
        base = sid * per_s

        def body(j, carry):
            off = base + j * CH
            pltpu.sync_copy(si.at[pl.ds(off, CH)], iv)
            pltpu.sync_copy(v.at[pl.ds(off, CH), pl.ds(c0, 128)], rv)
            pltpu.sync_copy(rv, acc.at[iv], add=True)
            return carry

        lax.fori_loop(0, steps, body, 0)
        plsc.subcore_barrier()

        def body2(j, carry):
            off = base + j * CH
            pltpu.sync_copy(gi.at[pl.ds(off, CH)], iv)
            pltpu.async_copy(acc.at[iv], rv, sem).wait()
            pltpu.sync_copy(rv, msg.at[pl.ds(off, CH), pl.ds(c0, 128)])
            return carry

        lax.fori_loop(0, steps, body2, 0)

    return mk


def _dot_t(x, w):
    return lax.dot_general(x, w, (((1,), (1,)), ((), ())),
                           preferred_element_type=_f32)


def _tc_gat_proj(h, wl, wr):
    def kfn(h_ref, wl_ref, wr_ref, xl_ref, xr_ref):
        hh = h_ref[...]
        xl_ref[...] = _dot_t(hh, wl_ref[...])
        xr_ref[...] = _dot_t(hh, wr_ref[...])

    n, d = h.shape
    return pl.pallas_call(
        kfn, out_shape=(_sds((n, wl.shape[0])), _sds((n, wr.shape[0]))),
    )(h, wl, wr)


def _tc_gat_edge(xls, xrd, att, n_real):
    E, D = xls.shape
    BLK = 2048
    nb = E // BLK
    att2 = jnp.zeros((D, D), _f32).at[0].set(att)

    def kfn(xl_ref, xr_ref, att_ref, p_ref):
        i = pl.program_id(0)
        j = pl.program_id(1)
        xl = xl_ref[...]
        u = xl + xr_ref[...]
        lr = jnp.where(u >= 0.0, u, 0.2 * u)
        e = _dot_t(lr, att_ref[...])[:, 0]
        rows = i * BLK + lax.broadcasted_iota(jnp.int32, (BLK,), 0)
        ex = jnp.where(rows < n_real, jnp.exp(e), 0.0)

        @pl.when(j == 0)
        def _():
            p_ref[...] = ex[:, None] * xl

        @pl.when(j == 1)
        def _():
            p_ref[...] = jnp.broadcast_to(ex[:, None], (BLK, D))

    return pl.pallas_call(
        kfn,
        grid=(nb, 2),
        in_specs=[
            pl.BlockSpec((BLK, D), lambda i, j: (i, 0)),
            pl.BlockSpec((BLK, D), lambda i, j: (i, 0)),
            pl.BlockSpec((D, D), lambda i, j: (0, 0)),
        ],
        out_specs=pl.BlockSpec((BLK, D), lambda i, j: (j * nb + i, 0)),
        out_shape=_sds((2 * E, D)),
    )(xls, xrd, att2)


def _tc_gat_norm(u, b, g, bb):
    n = N_NODES
    d = u.shape[2]

    def kfn(u_ref, b_ref, g_ref, bb_ref, out_ref):
        U = u_ref[0, :n, :]
        den = u_ref[1, :n, 0:1]
        y = U / (den + 1e-16) + b_ref[...]
        mu = jnp.mean(y, axis=0, keepdims=True)
        var = jnp.mean((y - mu) ** 2, axis=0, keepdims=True)
        yn = (y - mu) / jnp.sqrt(var + 1e-5) * g_ref[...] + bb_ref[...]
        out_ref[...] = jnp.maximum(yn, 0.0)

    return pl.pallas_call(
        kfn, out_shape=_sds((n, d)),
    )(u, b.reshape(1, d), g.reshape(1, d), bb.reshape(1, d))


def _tc_edge_proj(h1, h2, wl1, wl2, wr1, wr2, be):
    n = h1.shape[0]
    H = wl1.shape[0]
    BLK = 2000
    grid = n // BLK

    def kfn(h1_ref, h2_ref, wl1_ref, wl2_ref, wr1_ref, wr2_ref, be_ref,
            a_ref, b_ref):
        x1 = h1_ref[...]
        x2 = h2_ref[...]
        a_ref[...] = (_dot_t(x1, wl1_ref[...]) + _dot_t(x2, wl2_ref[...])
                      + be_ref[...])
        b_ref[...] = _dot_t(x1, wr1_ref[...]) + _dot_t(x2, wr2_ref[...])

    wspec = lambda shape: pl.BlockSpec(shape, lambda i: (0, 0))
    return pl.pallas_call(
        kfn,
        grid=(grid,),
        in_specs=[
            pl.BlockSpec((BLK, 128), lambda i: (i, 0)),
            pl.BlockSpec((BLK, 128), lambda i: (i, 0)),
            wspec(wl1.shape), wspec(wl2.shape),
            wspec(wr1.shape), wspec(wr2.shape),
            wspec((1, H)),
        ],
        out_specs=[
            pl.BlockSpec((BLK, H), lambda i: (i, 0)),
            pl.BlockSpec((BLK, H), lambda i: (i, 0)),
        ],
        out_shape=(_sds((n, H)), _sds((n, H))),
    )(h1, h2, wl1, wl2, wr1, wr2, be.reshape(1, H))


def _sigmoid(z):
    return 1.0 / (1.0 + jnp.exp(-z))


def _tc_ef_att(a, b, aw, ab, n_real):
    E, H = a.shape
    BLK = 2048
    grid = E // BLK

    def kfn(a_ref, b_ref, aw_ref, ab_ref, ef_ref, g_ref):
        i = pl.program_id(0)
        ef = a_ref[...] + b_ref[...]
        z = _dot_t(ef, aw_ref[...])[:, 0] + ab_ref[0, 0]
        att = _sigmoid(z)
        rows = i * BLK + lax.broadcasted_iota(jnp.int32, (BLK,), 0)
        att = jnp.where(rows < n_real, att, 0.0)
        ef_ref[...] = ef
        g_ref[...] = att[:, None] * ef

    wspec = lambda shape: pl.BlockSpec(shape, lambda i: (0, 0))
    return pl.pallas_call(
        kfn,
        grid=(grid,),
        in_specs=[
            pl.BlockSpec((BLK, H), lambda i: (i, 0)),
            pl.BlockSpec((BLK, H), lambda i: (i, 0)),
            wspec((128, H)), wspec((1, 1)),
        ],
        out_specs=[
            pl.BlockSpec((BLK, H), lambda i: (i, 0)),
            pl.BlockSpec((BLK, H), lambda i: (i, 0)),
        ],
        out_shape=(_sds((E, H)), _sds((E, H))),
    )(a, b, jnp.zeros((128, H), _f32).at[0].set(aw[0]), ab.reshape(1, 1))


def _tc_mp_mlp(ef, msg, w1, b1, w2, b2, aw, ab, n_real):
    E, H = ef.shape
    BLK = 2048
    grid = E // BLK

    def kfn(ef_ref, msg_ref, w1_ref, b1_ref, w2_ref, b2_ref, aw_ref, ab_ref,
            efn_ref, gn_ref):
        i = pl.program_id(0)
        h = ef_ref[...] + msg_ref[...]
        h2 = jnp.maximum(_dot_t(h, w1_ref[...]) + b1_ref[...], 0.0)
        efn = _dot_t(h2, w2_ref[...]) + b2_ref[...]
        z = _dot_t(efn, aw_ref[...])[:, 0] + ab_ref[0, 0]
        att = _sigmoid(z)
        rows = i * BLK + lax.broadcasted_iota(jnp.int32, (BLK,), 0)
        att = jnp.where(rows < n_real, att, 0.0)
        efn_ref[...] = efn
        gn_ref[...] = att[:, None] * efn

    wspec = lambda shape: pl.BlockSpec(shape, lambda i: (0, 0))
    return pl.pallas_call(
        kfn,
        grid=(grid,),
        in_specs=[
            pl.BlockSpec((BLK, H), lambda i: (i, 0)),
            pl.BlockSpec((BLK, H), lambda i: (i, 0)),
            wspec(w1.shape), wspec((1, H)),
            wspec(w2.shape), wspec((1, H)),
            wspec((128, H)), wspec((1, 1)),
        ],
        out_specs=[
            pl.BlockSpec((BLK, H), lambda i: (i, 0)),
            pl.BlockSpec((BLK, H), lambda i: (i, 0)),
        ],
        out_shape=(_sds((E, H)), _sds((E, H))),
    )(ef, msg, w1, b1.reshape(1, H), w2, b2.reshape(1, H),
      jnp.zeros((128, H), _f32).at[0].set(aw[0]), ab.reshape(1, 1))


def _tc_mp_final(ef, msg, w1, b1, w2, b2, rw1, rb1, rw2, rb2):
    E, H = ef.shape
    BLK = 2048
    grid = E // BLK

    def kfn(ef_ref, msg_ref, w1_ref, b1_ref, w2_ref, b2_ref,
            rw1_ref, rb1_ref, rw2_ref, rb2_ref, out_ref):
        h = ef_ref[...] + msg_ref[...]
        h2 = jnp.maximum(_dot_t(h, w1_ref[...]) + b1_ref[...], 0.0)
        ef2 = _dot_t(h2, w2_ref[...]) + b2_ref[...]
        h3 = jnp.maximum(_dot_t(ef2, rw1_ref[...]) + rb1_ref[...], 0.0)
        out_ref[...] = _dot_t(h3, rw2_ref[...])[:, 0:1] + rb2_ref[0, 0]

    wspec = lambda shape: pl.BlockSpec(shape, lambda i: (0, 0))
    return pl.pallas_call(
        kfn,
        grid=(grid,),
        in_specs=[
            pl.BlockSpec((BLK, H), lambda i: (i, 0)),
            pl.BlockSpec((BLK, H), lambda i: (i, 0)),
            wspec(w1.shape), wspec((1, H)),
            wspec(w2.shape), wspec((1, H)),
            wspec(rw1.shape), wspec((1, H)),
            wspec((128, H)), wspec((1, 1)),
        ],
        out_specs=pl.BlockSpec((BLK, 1), lambda i: (i, 0)),
        out_shape=_sds((E, 1)),
    )(ef, msg, w1, b1.reshape(1, H), w2, b2.reshape(1, H),
      rw1, rb1.reshape(1, H), jnp.zeros((128, H), _f32).at[0].set(rw2[0]),
      rb2.reshape(1, 1))


def _pad_idx(idx, Ep):
    return jnp.concatenate(
        [idx, jnp.zeros((Ep - idx.shape[0],), idx.dtype)])


def kernel(x, params, edge_index):
    p = params
    src, dst = edge_index[0], edge_index[1]
    n = x.shape[0]
    E = src.shape[0]
    loop = jnp.arange(n, dtype=src.dtype)

    ALIGN = CH * NW
    E1 = E + n
    E1p = ((E1 + ALIGN - 1) // ALIGN) * ALIGN
    E2p = ((E + ALIGN - 1) // ALIGN) * ALIGN

    s1 = _pad_idx(jnp.concatenate([src, loop]), E1p)
    d1 = _pad_idx(jnp.concatenate([dst, loop]), E1p)
    s2 = _pad_idx(src, E2p)
    d2 = _pad_idx(dst, E2p)
    z = jnp.zeros((NP, 128), _f32)

    h = x
    hs = []
    for li in range(2):
        wl, wr = p['conv%d_wl' % li], p['conv%d_wr' % li]
        xl, xr = _tc_gat_proj(h, wl, wr)
        xls, xrd = _sc_gather2(E1p, 128, 128)(xl, s1, xr, d1)
        P = _tc_gat_edge(xls, xrd, p['conv%d_att' % li], E1)
        u = _sc_scatter_gat(E1p)(P, d1, z)
        h = _tc_gat_norm(u, p['conv%d_b' % li],
                         p['bn%d_g' % li], p['bn%d_b' % li])
        hs.append(h)
    h1, h2 = hs

    we = p['edge_init_w']
    a_n, b_n = _tc_edge_proj(
        h1, h2,
        we[:, 0:128], we[:, 128:256], we[:, 256:384], we[:, 384:512],
        p['edge_init_b'])
    A, B = _sc_gather2(E2p, 256, 256)(a_n, s2, b_n, d2)

    ef, g0 = _tc_ef_att(A, B, p['mp0_att_w'], p['mp0_att_b'], E)
    msg0 = _sc_seg_msg(E2p)(g0, s2, s2, z)
    ef1, g1 = _tc_mp_mlp(ef, msg0, p['mp0_w1'], p['mp0_b1'],
                         p['mp0_w2'], p['mp0_b2'],
                         p['mp1_att_w'], p['mp1_att_b'], E)
    msg1 = _sc_seg_msg(E2p)(g1, s2, s2, z)
    out = _tc_mp_final(ef1, msg1, p['mp1_w1'], p['mp1_b1'],
                       p['mp1_w2'], p['mp1_b2'],
                       p['reg_w1'], p['reg_b1'], p['reg_w2'], p['reg_b2'])
    return out[:E]
